# baseline (device time: 53827 ns/iter reference)
import jax
import jax.numpy as jnp
from jax import lax
from jax.experimental import pallas as pl
from jax.experimental.pallas import tpu as pltpu

N_DEV = 16
EPS = 1e-5


def kernel(x, gamma):
    m, n_local = x.shape
    n_global = N_DEV * n_local
    gamma2d = gamma.reshape(1, n_local)

    def body(x_ref, g_ref, out_ref, comm_ref, send_sems, recv_sems):
        my_pos = lax.axis_index("i")

        xx = x_ref[...]
        partial = jnp.sum(xx * xx, axis=1, keepdims=True)
        comm_ref[pl.ds(my_pos, 1), :, :] = partial[None, :, :]

        for d in range(1, N_DEV):
            peer = lax.rem(my_pos + d, N_DEV)
            rdma = pltpu.make_async_remote_copy(
                src_ref=comm_ref.at[my_pos],
                dst_ref=comm_ref.at[my_pos],
                send_sem=send_sems.at[d - 1],
                recv_sem=recv_sems.at[my_pos],
                device_id=(peer,),
                device_id_type=pl.DeviceIdType.MESH,
            )
            rdma.start()

        for d in range(1, N_DEV):
            src = lax.rem(my_pos + d, N_DEV)
            recv = pltpu.make_async_remote_copy(
                src_ref=comm_ref.at[src],
                dst_ref=comm_ref.at[src],
                send_sem=send_sems.at[d - 1],
                recv_sem=recv_sems.at[src],
                device_id=(src,),
                device_id_type=pl.DeviceIdType.MESH,
            )
            recv.wait_recv()

        total = jnp.sum(comm_ref[...], axis=0)
        inv_rms = lax.rsqrt(total * (1.0 / n_global) + EPS)
        out_ref[...] = xx * g_ref[...] * inv_rms

        for d in range(1, N_DEV):
            peer = lax.rem(my_pos + d, N_DEV)
            send = pltpu.make_async_remote_copy(
                src_ref=comm_ref.at[my_pos],
                dst_ref=comm_ref.at[my_pos],
                send_sem=send_sems.at[d - 1],
                recv_sem=recv_sems.at[my_pos],
                device_id=(peer,),
                device_id_type=pl.DeviceIdType.MESH,
            )
            send.wait_send()

    return pl.pallas_call(
        body,
        out_shape=jax.ShapeDtypeStruct((m, n_local), x.dtype),
        in_specs=[
            pl.BlockSpec(memory_space=pltpu.VMEM),
            pl.BlockSpec(memory_space=pltpu.VMEM),
        ],
        out_specs=pl.BlockSpec(memory_space=pltpu.VMEM),
        scratch_shapes=[
            pltpu.VMEM((N_DEV, m, 1), jnp.float32),
            pltpu.SemaphoreType.DMA((N_DEV - 1,)),
            pltpu.SemaphoreType.DMA((N_DEV,)),
        ],
    )(x, gamma2d)


# device time: 11142 ns/iter; 4.8310x vs baseline; 4.8310x over previous
import jax
import jax.numpy as jnp
from jax import lax
from jax.experimental import pallas as pl
from jax.experimental.pallas import tpu as pltpu

N_DEV = 16
EPS = 1e-5


def kernel(x, gamma):
    m, n_local = x.shape
    n_global = N_DEV * n_local
    gamma2d = gamma.reshape(1, n_local)
    assert m % 128 == 0
    ms = m // 128

    def body(x_ref, g_ref, out_ref, comm_ref, send_sems, recv_sems):
        my_pos = lax.axis_index("i")

        xx = x_ref[...]
        partial = jnp.sum(
            xx.reshape(ms, 128, n_local) * xx.reshape(ms, 128, n_local), axis=2
        )
        comm_ref[pl.ds(my_pos, 1), :, :] = partial[None, :, :]

        barrier_sem = pltpu.get_barrier_semaphore()
        for d in range(1, N_DEV):
            peer = lax.rem(my_pos + d, N_DEV)
            pl.semaphore_signal(
                barrier_sem, inc=1,
                device_id=(peer,), device_id_type=pl.DeviceIdType.MESH,
            )
        pl.semaphore_wait(barrier_sem, N_DEV - 1)

        for d in range(1, N_DEV):
            peer = lax.rem(my_pos + d, N_DEV)
            rdma = pltpu.make_async_remote_copy(
                src_ref=comm_ref.at[my_pos],
                dst_ref=comm_ref.at[my_pos],
                send_sem=send_sems.at[d - 1],
                recv_sem=recv_sems.at[my_pos],
                device_id=(peer,),
                device_id_type=pl.DeviceIdType.MESH,
            )
            rdma.start()

        for d in range(1, N_DEV):
            src = lax.rem(my_pos + d, N_DEV)
            recv = pltpu.make_async_remote_copy(
                src_ref=comm_ref.at[src],
                dst_ref=comm_ref.at[src],
                send_sem=send_sems.at[d - 1],
                recv_sem=recv_sems.at[src],
                device_id=(src,),
                device_id_type=pl.DeviceIdType.MESH,
            )
            recv.wait_recv()

        total = jnp.sum(comm_ref[...], axis=0)
        inv_rms = lax.rsqrt(total * (1.0 / n_global) + EPS)
        eye = (
            lax.broadcasted_iota(jnp.int32, (128, 128), 0)
            == lax.broadcasted_iota(jnp.int32, (128, 128), 1)
        ).astype(jnp.float32)
        inv_col = jnp.concatenate(
            [
                jnp.sum(
                    jnp.broadcast_to(inv_rms[i : i + 1, :], (128, 128)) * eye,
                    axis=1,
                    keepdims=True,
                )
                for i in range(ms)
            ],
            axis=0,
        )
        out_ref[...] = xx * g_ref[...] * inv_col

        for d in range(1, N_DEV):
            peer = lax.rem(my_pos + d, N_DEV)
            send = pltpu.make_async_remote_copy(
                src_ref=comm_ref.at[my_pos],
                dst_ref=comm_ref.at[my_pos],
                send_sem=send_sems.at[d - 1],
                recv_sem=recv_sems.at[my_pos],
                device_id=(peer,),
                device_id_type=pl.DeviceIdType.MESH,
            )
            send.wait_send()

    return pl.pallas_call(
        body,
        out_shape=jax.ShapeDtypeStruct((m, n_local), x.dtype),
        in_specs=[
            pl.BlockSpec(memory_space=pltpu.VMEM),
            pl.BlockSpec(memory_space=pltpu.VMEM),
        ],
        out_specs=pl.BlockSpec(memory_space=pltpu.VMEM),
        scratch_shapes=[
            pltpu.VMEM((N_DEV, ms, 128), jnp.float32),
            pltpu.SemaphoreType.DMA((N_DEV - 1,)),
            pltpu.SemaphoreType.DMA((N_DEV,)),
        ],
        compiler_params=pltpu.CompilerParams(collective_id=0),
    )(x, gamma2d)


# device time: 10588 ns/iter; 5.0838x vs baseline; 1.0523x over previous
import jax
import jax.numpy as jnp
from jax import lax
from jax.experimental import pallas as pl
from jax.experimental.pallas import tpu as pltpu

N_DEV = 16
EPS = 1e-5


def kernel(x, gamma):
    m, n_local = x.shape
    n_global = N_DEV * n_local
    gamma2d = gamma.reshape(1, n_local)
    assert m % 128 == 0
    ms = m // 128

    def body(x_ref, g_ref, out_ref, comm_ref, send_sems, recv_sems):
        my_pos = lax.axis_index("i")

        barrier_sem = pltpu.get_barrier_semaphore()
        for d in range(1, N_DEV):
            peer = lax.rem(my_pos + d, N_DEV)
            pl.semaphore_signal(
                barrier_sem, inc=1,
                device_id=(peer,), device_id_type=pl.DeviceIdType.MESH,
            )

        xx = x_ref[...]
        x3 = xx.reshape(ms, 128, n_local)
        partial = jnp.sum(x3 * x3, axis=2)
        comm_ref[pl.ds(my_pos, 1), :, :] = partial[None, :, :]

        pl.semaphore_wait(barrier_sem, N_DEV - 1)

        for d in range(1, N_DEV):
            peer = lax.rem(my_pos + d, N_DEV)
            rdma = pltpu.make_async_remote_copy(
                src_ref=comm_ref.at[my_pos],
                dst_ref=comm_ref.at[my_pos],
                send_sem=send_sems.at[d - 1],
                recv_sem=recv_sems.at[my_pos],
                device_id=(peer,),
                device_id_type=pl.DeviceIdType.MESH,
            )
            rdma.start()

        xg = xx * g_ref[...]
        eye = (
            lax.broadcasted_iota(jnp.int32, (128, 128), 0)
            == lax.broadcasted_iota(jnp.int32, (128, 128), 1)
        ).astype(jnp.float32)

        for d in range(1, N_DEV):
            src = lax.rem(my_pos + d, N_DEV)
            recv = pltpu.make_async_remote_copy(
                src_ref=comm_ref.at[src],
                dst_ref=comm_ref.at[src],
                send_sem=send_sems.at[d - 1],
                recv_sem=recv_sems.at[src],
                device_id=(src,),
                device_id_type=pl.DeviceIdType.MESH,
            )
            recv.wait_recv()

        total = jnp.sum(comm_ref[...], axis=0)
        inv_rms = lax.rsqrt(total * (1.0 / n_global) + EPS)
        inv_col = jnp.concatenate(
            [
                jnp.sum(
                    jnp.broadcast_to(inv_rms[i : i + 1, :], (128, 128)) * eye,
                    axis=1,
                    keepdims=True,
                )
                for i in range(ms)
            ],
            axis=0,
        )
        out_ref[...] = xg * inv_col

        for d in range(1, N_DEV):
            peer = lax.rem(my_pos + d, N_DEV)
            send = pltpu.make_async_remote_copy(
                src_ref=comm_ref.at[my_pos],
                dst_ref=comm_ref.at[my_pos],
                send_sem=send_sems.at[d - 1],
                recv_sem=recv_sems.at[my_pos],
                device_id=(peer,),
                device_id_type=pl.DeviceIdType.MESH,
            )
            send.wait_send()

    return pl.pallas_call(
        body,
        out_shape=jax.ShapeDtypeStruct((m, n_local), x.dtype),
        in_specs=[
            pl.BlockSpec(memory_space=pltpu.VMEM),
            pl.BlockSpec(memory_space=pltpu.VMEM),
        ],
        out_specs=pl.BlockSpec(memory_space=pltpu.VMEM),
        scratch_shapes=[
            pltpu.VMEM((N_DEV, ms, 128), jnp.float32),
            pltpu.SemaphoreType.DMA((N_DEV - 1,)),
            pltpu.SemaphoreType.DMA((N_DEV,)),
        ],
        compiler_params=pltpu.CompilerParams(collective_id=0),
    )(x, gamma2d)
